# initial kernel scaffold (unmeasured)
import jax
import jax.numpy as jnp
from jax import lax
from jax.experimental import pallas as pl
from jax.experimental.pallas import tpu as pltpu

N_DEV = 4


def _ring_reduce_scatter_silu(partial):
    m, n = partial.shape
    m_per = m // N_DEV

    def body(p_ref, out_ref, comm, own, send_sems, recv_sems, local_sems):
        my = lax.axis_index("i")
        left = lax.rem(my + (N_DEV - 1), N_DEV)
        right = lax.rem(my + 1, N_DEV)

        c0 = lax.rem(my + (N_DEV - 1), N_DEV)
        ld = pltpu.make_async_copy(
            p_ref.at[pl.ds(c0 * m_per, m_per)], comm.at[0], local_sems.at[0]
        )
        ld.start()
        ld.wait()

        barrier_sem = pltpu.get_barrier_semaphore()
        for nbr in [left, right]:
            pl.semaphore_signal(
                barrier_sem, inc=1,
                device_id=(nbr,), device_id_type=pl.DeviceIdType.MESH,
            )
        pl.semaphore_wait(barrier_sem, 2)

        for h in range(N_DEV - 1):
            send_slot = h % 2
            recv_slot = (h + 1) % 2
            rdma = pltpu.make_async_remote_copy(
                src_ref=comm.at[send_slot],
                dst_ref=comm.at[recv_slot],
                send_sem=send_sems.at[h],
                recv_sem=recv_sems.at[h],
                device_id=(right,),
                device_id_type=pl.DeviceIdType.MESH,
            )
            rdma.start()
            c = lax.rem(my + (N_DEV - 2 - h), N_DEV)
            ld = pltpu.make_async_copy(
                p_ref.at[pl.ds(c * m_per, m_per)], own, local_sems.at[1]
            )
            ld.start()
            ld.wait()
            rdma.wait()
            if h < N_DEV - 2:
                comm[recv_slot] = comm[recv_slot] + own[...]
            else:
                y = comm[recv_slot] + own[...]
                own[...] = y * jax.nn.sigmoid(y)
                st = pltpu.make_async_copy(own, out_ref, local_sems.at[0])
                st.start()
                st.wait()

    return pl.pallas_call(
        body,
        out_shape=jax.ShapeDtypeStruct((m_per, n), jnp.float32),
        in_specs=[pl.BlockSpec(memory_space=pltpu.ANY)],
        out_specs=pl.BlockSpec(memory_space=pltpu.ANY),
        scratch_shapes=[
            pltpu.VMEM((2, m_per, n), jnp.float32),
            pltpu.VMEM((m_per, n), jnp.float32),
            pltpu.SemaphoreType.DMA((N_DEV - 1,)),
            pltpu.SemaphoreType.DMA((N_DEV - 1,)),
            pltpu.SemaphoreType.DMA((2,)),
        ],
        compiler_params=pltpu.CompilerParams(collective_id=0),
    )(partial)


def kernel(x, w_mat):
    partial = jnp.dot(
        x, w_mat,
        preferred_element_type=jnp.float32,
        precision=lax.Precision.HIGHEST,
    )
    return _ring_reduce_scatter_silu(partial)


# baseline (device time: 2147643 ns/iter reference)
import jax
import jax.numpy as jnp
from jax import lax
from jax.experimental import pallas as pl
from jax.experimental.pallas import tpu as pltpu

N_DEV = 4
N_SEG = 2


def _ring_reduce_scatter_silu(partial):
    m, n = partial.shape
    m_per = m // N_DEV
    m_seg = m_per // N_SEG

    def body(p_ref, out_ref, comm, own, send_sems, recv_sems, local_sems):
        my = lax.axis_index("i")
        left = lax.rem(my + (N_DEV - 1), N_DEV)
        right = lax.rem(my + 1, N_DEV)
        barrier_sem = pltpu.get_barrier_semaphore()

        def nbr_barrier():
            for nbr in [left, right]:
                pl.semaphore_signal(
                    barrier_sem, inc=1,
                    device_id=(nbr,), device_id_type=pl.DeviceIdType.MESH,
                )
            pl.semaphore_wait(barrier_sem, 2)

        for s in range(N_SEG):
            c0 = lax.rem(my + (N_DEV - 1), N_DEV)
            ld = pltpu.make_async_copy(
                p_ref.at[pl.ds(c0 * m_per + s * m_seg, m_seg)],
                comm.at[0],
                local_sems.at[0],
            )
            ld.start()
            ld.wait()

            nbr_barrier()

            for h in range(N_DEV - 1):
                send_slot = h % 2
                recv_slot = (h + 1) % 2
                k = s * (N_DEV - 1) + h
                rdma = pltpu.make_async_remote_copy(
                    src_ref=comm.at[send_slot],
                    dst_ref=comm.at[recv_slot],
                    send_sem=send_sems.at[k],
                    recv_sem=recv_sems.at[k],
                    device_id=(right,),
                    device_id_type=pl.DeviceIdType.MESH,
                )
                rdma.start()
                c = lax.rem(my + (N_DEV - 2 - h), N_DEV)
                ld = pltpu.make_async_copy(
                    p_ref.at[pl.ds(c * m_per + s * m_seg, m_seg)],
                    own,
                    local_sems.at[1],
                )
                ld.start()
                ld.wait()
                rdma.wait()
                if h < N_DEV - 2:
                    comm[recv_slot] = comm[recv_slot] + own[...]
                else:
                    y = comm[recv_slot] + own[...]
                    own[...] = y * jax.nn.sigmoid(y)
                    st = pltpu.make_async_copy(
                        own,
                        out_ref.at[pl.ds(s * m_seg, m_seg)],
                        local_sems.at[0],
                    )
                    st.start()
                    st.wait()

    return pl.pallas_call(
        body,
        out_shape=jax.ShapeDtypeStruct((m_per, n), jnp.float32),
        in_specs=[pl.BlockSpec(memory_space=pl.ANY)],
        out_specs=pl.BlockSpec(memory_space=pl.ANY),
        scratch_shapes=[
            pltpu.VMEM((2, m_seg, n), jnp.float32),
            pltpu.VMEM((m_seg, n), jnp.float32),
            pltpu.SemaphoreType.DMA((N_SEG * (N_DEV - 1),)),
            pltpu.SemaphoreType.DMA((N_SEG * (N_DEV - 1),)),
            pltpu.SemaphoreType.DMA((2,)),
        ],
        compiler_params=pltpu.CompilerParams(
            collective_id=0,
            vmem_limit_bytes=60 * 1024 * 1024,
        ),
    )(partial)


def kernel(x, w_mat):
    partial = jnp.dot(
        x, w_mat,
        preferred_element_type=jnp.float32,
        precision=lax.Precision.HIGHEST,
    )
    return _ring_reduce_scatter_silu(partial)


# device time: 803489 ns/iter; 2.6729x vs baseline; 2.6729x over previous
import jax
import jax.numpy as jnp
from jax import lax
from jax.experimental import pallas as pl
from jax.experimental.pallas import tpu as pltpu

N_DEV = 4
N_ROUND = 2


def _ring_reduce_scatter_silu(partial):
    m, n = partial.shape
    m_per = m // N_DEV
    m_seg = m_per // (2 * N_ROUND)
    n_hops = N_DEV - 1

    def body(p_ref, out_ref,
             comm_cw, comm_ccw, own_cw, own_ccw,
             cw_send, cw_recv, ccw_send, ccw_recv, local_sems):
        my = lax.axis_index("i")
        left = lax.rem(my + (N_DEV - 1), N_DEV)
        right = lax.rem(my + 1, N_DEV)
        barrier_sem = pltpu.get_barrier_semaphore()

        def nbr_barrier():
            for nbr in [left, right]:
                pl.semaphore_signal(
                    barrier_sem, inc=1,
                    device_id=(nbr,), device_id_type=pl.DeviceIdType.MESH,
                )
            pl.semaphore_wait(barrier_sem, 2)

        def row(c, q):
            return c * m_per + q * m_seg

        for r in range(N_ROUND):
            q_cw = r
            q_ccw = N_ROUND + r

            ld0 = pltpu.make_async_copy(
                p_ref.at[pl.ds(row(left, q_cw), m_seg)],
                comm_cw.at[0], local_sems.at[0])
            ld1 = pltpu.make_async_copy(
                p_ref.at[pl.ds(row(right, q_ccw), m_seg)],
                comm_ccw.at[0], local_sems.at[1])
            ld0.start()
            ld1.start()
            ld0.wait()
            ld1.wait()

            nbr_barrier()

            for h in range(n_hops):
                send_slot = h % 2
                recv_slot = (h + 1) % 2
                k = r * n_hops + h
                rdma_cw = pltpu.make_async_remote_copy(
                    src_ref=comm_cw.at[send_slot],
                    dst_ref=comm_cw.at[recv_slot],
                    send_sem=cw_send.at[k],
                    recv_sem=cw_recv.at[k],
                    device_id=(right,),
                    device_id_type=pl.DeviceIdType.MESH,
                )
                rdma_ccw = pltpu.make_async_remote_copy(
                    src_ref=comm_ccw.at[send_slot],
                    dst_ref=comm_ccw.at[recv_slot],
                    send_sem=ccw_send.at[k],
                    recv_sem=ccw_recv.at[k],
                    device_id=(left,),
                    device_id_type=pl.DeviceIdType.MESH,
                )
                rdma_cw.start()
                rdma_ccw.start()

                c_cw = lax.rem(my + (N_DEV - 2 - h), N_DEV)
                c_ccw = lax.rem(my + (h + 2), N_DEV)
                ld0 = pltpu.make_async_copy(
                    p_ref.at[pl.ds(row(c_cw, q_cw), m_seg)],
                    own_cw, local_sems.at[0])
                ld1 = pltpu.make_async_copy(
                    p_ref.at[pl.ds(row(c_ccw, q_ccw), m_seg)],
                    own_ccw, local_sems.at[1])
                ld0.start()
                ld1.start()
                ld0.wait()
                ld1.wait()

                rdma_cw.wait()
                rdma_ccw.wait()

                if h < n_hops - 1:
                    comm_cw[recv_slot] = comm_cw[recv_slot] + own_cw[...]
                    comm_ccw[recv_slot] = comm_ccw[recv_slot] + own_ccw[...]
                else:
                    y = comm_cw[recv_slot] + own_cw[...]
                    own_cw[...] = y * jax.nn.sigmoid(y)
                    z = comm_ccw[recv_slot] + own_ccw[...]
                    own_ccw[...] = z * jax.nn.sigmoid(z)
                    st0 = pltpu.make_async_copy(
                        own_cw, out_ref.at[pl.ds(q_cw * m_seg, m_seg)],
                        local_sems.at[0])
                    st1 = pltpu.make_async_copy(
                        own_ccw, out_ref.at[pl.ds(q_ccw * m_seg, m_seg)],
                        local_sems.at[1])
                    st0.start()
                    st1.start()
                    st0.wait()
                    st1.wait()

    n_sem = N_ROUND * n_hops
    return pl.pallas_call(
        body,
        out_shape=jax.ShapeDtypeStruct((m_per, n), jnp.float32),
        in_specs=[pl.BlockSpec(memory_space=pl.ANY)],
        out_specs=pl.BlockSpec(memory_space=pl.ANY),
        scratch_shapes=[
            pltpu.VMEM((2, m_seg, n), jnp.float32),
            pltpu.VMEM((2, m_seg, n), jnp.float32),
            pltpu.VMEM((m_seg, n), jnp.float32),
            pltpu.VMEM((m_seg, n), jnp.float32),
            pltpu.SemaphoreType.DMA((n_sem,)),
            pltpu.SemaphoreType.DMA((n_sem,)),
            pltpu.SemaphoreType.DMA((n_sem,)),
            pltpu.SemaphoreType.DMA((n_sem,)),
            pltpu.SemaphoreType.DMA((2,)),
        ],
        compiler_params=pltpu.CompilerParams(
            collective_id=0,
            vmem_limit_bytes=60 * 1024 * 1024,
        ),
    )(partial)


def kernel(x, w_mat):
    partial = jnp.dot(x, w_mat, preferred_element_type=jnp.float32)
    return _ring_reduce_scatter_silu(partial)


# device time: 698189 ns/iter; 3.0760x vs baseline; 1.1508x over previous
import jax
import jax.numpy as jnp
from jax import lax
from jax.experimental import pallas as pl
from jax.experimental.pallas import tpu as pltpu

N_DEV = 4
N_ROUND = 2
K_BLK = 128


def kernel(x, w_mat):
    m, k = x.shape
    _, n = w_mat.shape
    m_per = m // N_DEV
    m_seg = m_per // (2 * N_ROUND)
    n_hops = N_DEV - 1
    n_kblk = k // K_BLK

    def body(x_ref, w_ref, out_ref,
             comm_cw, comm_ccw, acc_cw, acc_ccw, x_cw, x_ccw, w_stage,
             cw_send, cw_recv, ccw_send, ccw_recv, x_sems, w_sems):
        my = lax.axis_index("i")
        left = lax.rem(my + (N_DEV - 1), N_DEV)
        right = lax.rem(my + 1, N_DEV)
        barrier_sem = pltpu.get_barrier_semaphore()

        def nbr_barrier():
            for nbr in [left, right]:
                pl.semaphore_signal(
                    barrier_sem, inc=1,
                    device_id=(nbr,), device_id_type=pl.DeviceIdType.MESH,
                )
            pl.semaphore_wait(barrier_sem, 2)

        def row(c, q):
            return c * m_per + q * m_seg

        def load_x(c_cw, q_cw, c_ccw, q_ccw):
            l0 = pltpu.make_async_copy(
                x_ref.at[pl.ds(row(c_cw, q_cw), m_seg)], x_cw, x_sems.at[0])
            l1 = pltpu.make_async_copy(
                x_ref.at[pl.ds(row(c_ccw, q_ccw), m_seg)], x_ccw, x_sems.at[1])
            l0.start()
            l1.start()
            return l0, l1

        def w_copy(kb, slot):
            return pltpu.make_async_copy(
                w_ref.at[pl.ds(kb * K_BLK, K_BLK)],
                w_stage.at[slot], w_sems.at[slot])

        def gemm_both(dst0, dst1):
            dst0[...] = jnp.zeros_like(dst0)
            dst1[...] = jnp.zeros_like(dst1)
            w_copy(0, 0).start()

            def kbody(kb, _):
                slot = lax.rem(kb, 2)

                @pl.when(kb + 1 < n_kblk)
                def _():
                    w_copy(kb + 1, lax.rem(kb + 1, 2)).start()

                w_copy(kb, slot).wait()
                xs = pl.ds(kb * K_BLK, K_BLK)
                p0 = jnp.dot(x_cw[:, xs], w_stage[slot],
                             preferred_element_type=jnp.float32)
                p1 = jnp.dot(x_ccw[:, xs], w_stage[slot],
                             preferred_element_type=jnp.float32)
                dst0[...] = dst0[...] + p0
                dst1[...] = dst1[...] + p1
                return 0

            lax.fori_loop(0, n_kblk, kbody, 0)

        def round_body(r, _):
            q_cw = r
            q_ccw = N_ROUND + r

            l0, l1 = load_x(left, q_cw, right, q_ccw)
            l0.wait()
            l1.wait()
            gemm_both(comm_cw.at[0], comm_ccw.at[0])

            nbr_barrier()

            def hop_body(h, _):
                send_slot = lax.rem(h, 2)
                recv_slot = lax.rem(h + 1, 2)
                sk = r * n_hops + h
                rdma_cw = pltpu.make_async_remote_copy(
                    src_ref=comm_cw.at[send_slot],
                    dst_ref=comm_cw.at[recv_slot],
                    send_sem=cw_send.at[sk],
                    recv_sem=cw_recv.at[sk],
                    device_id=(right,),
                    device_id_type=pl.DeviceIdType.MESH,
                )
                rdma_ccw = pltpu.make_async_remote_copy(
                    src_ref=comm_ccw.at[send_slot],
                    dst_ref=comm_ccw.at[recv_slot],
                    send_sem=ccw_send.at[sk],
                    recv_sem=ccw_recv.at[sk],
                    device_id=(left,),
                    device_id_type=pl.DeviceIdType.MESH,
                )
                rdma_cw.start()
                rdma_ccw.start()

                c_cw = lax.rem(my + (N_DEV - 2) - h, N_DEV)
                c_ccw = lax.rem(my + h + 2, N_DEV)
                l0, l1 = load_x(c_cw, q_cw, c_ccw, q_ccw)
                l0.wait()
                l1.wait()
                gemm_both(acc_cw, acc_ccw)

                rdma_cw.wait()
                rdma_ccw.wait()

                @pl.when(h < n_hops - 1)
                def _():
                    comm_cw[recv_slot] = comm_cw[recv_slot] + acc_cw[...]
                    comm_ccw[recv_slot] = comm_ccw[recv_slot] + acc_ccw[...]

                @pl.when(h == n_hops - 1)
                def _():
                    y = comm_cw[recv_slot] + acc_cw[...]
                    acc_cw[...] = y * jax.nn.sigmoid(y)
                    z = comm_ccw[recv_slot] + acc_ccw[...]
                    acc_ccw[...] = z * jax.nn.sigmoid(z)
                    st0 = pltpu.make_async_copy(
                        acc_cw, out_ref.at[pl.ds(q_cw * m_seg, m_seg)],
                        x_sems.at[0])
                    st1 = pltpu.make_async_copy(
                        acc_ccw, out_ref.at[pl.ds(q_ccw * m_seg, m_seg)],
                        x_sems.at[1])
                    st0.start()
                    st1.start()
                    st0.wait()
                    st1.wait()

                return 0

            lax.fori_loop(0, n_hops, hop_body, 0)
            return 0

        lax.fori_loop(0, N_ROUND, round_body, 0)

    n_sem = N_ROUND * n_hops
    return pl.pallas_call(
        body,
        out_shape=jax.ShapeDtypeStruct((m_per, n), jnp.float32),
        in_specs=[pl.BlockSpec(memory_space=pl.ANY),
                  pl.BlockSpec(memory_space=pl.ANY)],
        out_specs=pl.BlockSpec(memory_space=pl.ANY),
        scratch_shapes=[
            pltpu.VMEM((2, m_seg, n), jnp.float32),
            pltpu.VMEM((2, m_seg, n), jnp.float32),
            pltpu.VMEM((m_seg, n), jnp.float32),
            pltpu.VMEM((m_seg, n), jnp.float32),
            pltpu.VMEM((m_seg, k), jnp.float32),
            pltpu.VMEM((m_seg, k), jnp.float32),
            pltpu.VMEM((2, K_BLK, n), jnp.float32),
            pltpu.SemaphoreType.DMA((n_sem,)),
            pltpu.SemaphoreType.DMA((n_sem,)),
            pltpu.SemaphoreType.DMA((n_sem,)),
            pltpu.SemaphoreType.DMA((n_sem,)),
            pltpu.SemaphoreType.DMA((2,)),
            pltpu.SemaphoreType.DMA((2,)),
        ],
        compiler_params=pltpu.CompilerParams(
            collective_id=0,
            vmem_limit_bytes=62 * 1024 * 1024,
        ),
    )(x, w_mat)


# device time: 391244 ns/iter; 5.4893x vs baseline; 1.7845x over previous
import jax
import jax.numpy as jnp
from jax import lax
from jax.experimental import pallas as pl
from jax.experimental.pallas import tpu as pltpu

N_DEV = 4
N_ROUND = 2
K_BLK = 256


def kernel(x, w_mat):
    m, k = x.shape
    _, n = w_mat.shape
    m_per = m // N_DEV
    m_seg = m_per // (2 * N_ROUND)
    n_hops = N_DEV - 1
    n_kblk = k // K_BLK

    def body(x_ref, w_ref, out_ref,
             comm_cw, comm_ccw, acc_cw, acc_ccw, x_cw, x_ccw, w_stage,
             cw_send, cw_recv, ccw_send, ccw_recv, x_sems, w_sems):
        my = lax.axis_index("i")
        left = lax.rem(my + (N_DEV - 1), N_DEV)
        right = lax.rem(my + 1, N_DEV)
        barrier_sem = pltpu.get_barrier_semaphore()

        def nbr_barrier():
            for nbr in [left, right]:
                pl.semaphore_signal(
                    barrier_sem, inc=1,
                    device_id=(nbr,), device_id_type=pl.DeviceIdType.MESH,
                )
            pl.semaphore_wait(barrier_sem, 2)

        def row(c, q):
            return c * m_per + q * m_seg

        def load_x(c_cw, q_cw, c_ccw, q_ccw):
            l0 = pltpu.make_async_copy(
                x_ref.at[pl.ds(row(c_cw, q_cw), m_seg)], x_cw, x_sems.at[0])
            l1 = pltpu.make_async_copy(
                x_ref.at[pl.ds(row(c_ccw, q_ccw), m_seg)], x_ccw, x_sems.at[1])
            l0.start()
            l1.start()
            return l0, l1

        def w_copy(kb, slot):
            return pltpu.make_async_copy(
                w_ref.at[pl.ds(kb * K_BLK, K_BLK)],
                w_stage.at[slot], w_sems.at[slot])

        def gemm_both(dst0, dst1):
            dst0[...] = jnp.zeros_like(dst0)
            dst1[...] = jnp.zeros_like(dst1)
            w_copy(0, 0).start()

            def kbody(kb, _):
                slot = lax.rem(kb, 2)

                @pl.when(kb + 1 < n_kblk)
                def _():
                    w_copy(kb + 1, lax.rem(kb + 1, 2)).start()

                w_copy(kb, slot).wait()
                xs = pl.ds(kb * K_BLK, K_BLK)
                p0 = jnp.dot(x_cw[:, xs], w_stage[slot],
                             preferred_element_type=jnp.float32)
                p1 = jnp.dot(x_ccw[:, xs], w_stage[slot],
                             preferred_element_type=jnp.float32)
                dst0[...] = dst0[...] + p0
                dst1[...] = dst1[...] + p1
                return 0

            lax.fori_loop(0, n_kblk, kbody, 0)

        def round_body(r, _):
            q_cw = r
            q_ccw = N_ROUND + r

            l0, l1 = load_x(left, q_cw, right, q_ccw)
            l0.wait()
            l1.wait()
            gemm_both(acc_cw, acc_ccw)
            comm_cw[0] = acc_cw[...].astype(jnp.bfloat16)
            comm_ccw[0] = acc_ccw[...].astype(jnp.bfloat16)

            nbr_barrier()

            def hop_body(h, _):
                send_slot = lax.rem(h, 2)
                recv_slot = lax.rem(h + 1, 2)
                sk = r * n_hops + h
                rdma_cw = pltpu.make_async_remote_copy(
                    src_ref=comm_cw.at[send_slot],
                    dst_ref=comm_cw.at[recv_slot],
                    send_sem=cw_send.at[sk],
                    recv_sem=cw_recv.at[sk],
                    device_id=(right,),
                    device_id_type=pl.DeviceIdType.MESH,
                )
                rdma_ccw = pltpu.make_async_remote_copy(
                    src_ref=comm_ccw.at[send_slot],
                    dst_ref=comm_ccw.at[recv_slot],
                    send_sem=ccw_send.at[sk],
                    recv_sem=ccw_recv.at[sk],
                    device_id=(left,),
                    device_id_type=pl.DeviceIdType.MESH,
                )
                rdma_cw.start()
                rdma_ccw.start()

                c_cw = lax.rem(my + (N_DEV - 2) - h, N_DEV)
                c_ccw = lax.rem(my + h + 2, N_DEV)
                l0, l1 = load_x(c_cw, q_cw, c_ccw, q_ccw)
                l0.wait()
                l1.wait()
                gemm_both(acc_cw, acc_ccw)

                rdma_cw.wait()
                rdma_ccw.wait()

                @pl.when(h < n_hops - 1)
                def _():
                    comm_cw[recv_slot] = (
                        comm_cw[recv_slot].astype(jnp.float32) + acc_cw[...]
                    ).astype(jnp.bfloat16)
                    comm_ccw[recv_slot] = (
                        comm_ccw[recv_slot].astype(jnp.float32) + acc_ccw[...]
                    ).astype(jnp.bfloat16)

                @pl.when(h == n_hops - 1)
                def _():
                    y = comm_cw[recv_slot].astype(jnp.float32) + acc_cw[...]
                    acc_cw[...] = y * jax.nn.sigmoid(y)
                    z = comm_ccw[recv_slot].astype(jnp.float32) + acc_ccw[...]
                    acc_ccw[...] = z * jax.nn.sigmoid(z)
                    st0 = pltpu.make_async_copy(
                        acc_cw, out_ref.at[pl.ds(q_cw * m_seg, m_seg)],
                        x_sems.at[0])
                    st1 = pltpu.make_async_copy(
                        acc_ccw, out_ref.at[pl.ds(q_ccw * m_seg, m_seg)],
                        x_sems.at[1])
                    st0.start()
                    st1.start()
                    st0.wait()
                    st1.wait()

                return 0

            lax.fori_loop(0, n_hops, hop_body, 0)
            return 0

        lax.fori_loop(0, N_ROUND, round_body, 0)

    n_sem = N_ROUND * n_hops
    return pl.pallas_call(
        body,
        out_shape=jax.ShapeDtypeStruct((m_per, n), jnp.float32),
        in_specs=[pl.BlockSpec(memory_space=pl.ANY),
                  pl.BlockSpec(memory_space=pl.ANY)],
        out_specs=pl.BlockSpec(memory_space=pl.ANY),
        scratch_shapes=[
            pltpu.VMEM((2, m_seg, n), jnp.bfloat16),
            pltpu.VMEM((2, m_seg, n), jnp.bfloat16),
            pltpu.VMEM((m_seg, n), jnp.float32),
            pltpu.VMEM((m_seg, n), jnp.float32),
            pltpu.VMEM((m_seg, k), jnp.float32),
            pltpu.VMEM((m_seg, k), jnp.float32),
            pltpu.VMEM((2, K_BLK, n), jnp.float32),
            pltpu.SemaphoreType.DMA((n_sem,)),
            pltpu.SemaphoreType.DMA((n_sem,)),
            pltpu.SemaphoreType.DMA((n_sem,)),
            pltpu.SemaphoreType.DMA((n_sem,)),
            pltpu.SemaphoreType.DMA((2,)),
            pltpu.SemaphoreType.DMA((2,)),
        ],
        compiler_params=pltpu.CompilerParams(
            collective_id=0,
            vmem_limit_bytes=62 * 1024 * 1024,
        ),
    )(x, w_mat)
